# baseline (speedup 1.0000x reference)
import functools

import jax
import jax.numpy as jnp
from jax import lax
from jax.experimental import pallas as pl
from jax.experimental.pallas import tpu as pltpu
from jax.experimental.pallas import tpu_sc as plsc

_NC = 2
_NS = 16
_NW = _NC * _NS

_CHUNK = 400
_HALF = _CHUNK // 2
_NBUF = 4


def _embed_lookup(idx3, table):
    nw, n_steps, chunk = idx3.shape
    _, d = table.shape
    b_per_w = n_steps * chunk
    n = nw * b_per_w
    half = chunk // 2
    n_groups = n_steps // _NBUF
    mesh = plsc.VectorSubcoreMesh(core_axis_name="c", subcore_axis_name="s")

    @functools.partial(
        pl.kernel,
        mesh=mesh,
        out_type=jax.ShapeDtypeStruct((n // 2, 2 * d), jnp.float32),
        scratch_types=[
            pltpu.VMEM((n_steps, chunk), jnp.int32),
            pltpu.VMEM((_NBUF, chunk, d), jnp.float32),
            pltpu.SemaphoreType.DMA((_NBUF,)),
            pltpu.SemaphoreType.DMA((_NBUF,)),
        ],
        compiler_params=pltpu.CompilerParams(use_tc_tiling_on_sc=False),
    )
    def k(idx_hbm, tab_hbm, out_hbm, idx_all, rows, gsem, ssem):
        wid = lax.axis_index("s") * _NC + lax.axis_index("c")
        base2 = wid * (b_per_w // 2)

        def gather_start(b, i):
            pltpu.async_copy(tab_hbm.at[idx_all.at[i]], rows.at[b], gsem.at[b])

        def gather_wait(b, i):
            pltpu.make_async_copy(
                tab_hbm.at[idx_all.at[i]], rows.at[b], gsem.at[b]).wait()

        def _store_descs(b, i):
            r0 = base2 + i * half
            dst = out_hbm.at[pl.ds(r0, half)]
            return (
                (rows.at[b].at[pl.ds(0, half)], dst.at[:, pl.ds(0, d)]),
                (rows.at[b].at[pl.ds(half, half)], dst.at[:, pl.ds(d, d)]),
            )

        def store_start(b, i):
            for src, dst in _store_descs(b, i):
                pltpu.async_copy(src, dst, ssem.at[b])

        def store_wait(b, i):
            for src, dst in _store_descs(b, i):
                pltpu.make_async_copy(src, dst, ssem.at[b]).wait()

        pltpu.sync_copy(idx_hbm.at[wid], idx_all)

        for b in range(_NBUF):
            gather_start(b, b)
        gather_wait(0, 0)
        store_start(0, 0)

        def group(g, carry):
            for b in range(_NBUF):
                i = g * _NBUF + b
                store_wait(b, i - _NBUF)
                gather_start(b, i)
                kk = i - (_NBUF - 1)
                bk = (b + 1) % _NBUF
                gather_wait(bk, kk)
                store_start(bk, kk)
            return carry

        lax.fori_loop(1, n_groups, group, 0, unroll=False)

        for kk in range(n_steps - (_NBUF - 1), n_steps):
            gather_wait(kk % _NBUF, kk)
            store_start(kk % _NBUF, kk)
        for b in range(_NBUF):
            store_wait(b, n_steps - _NBUF + b)

    return k(idx3, table)


def kernel(sentence, word_embeds):
    b, s = sentence.shape
    d = word_embeds.shape[1]
    n = b * s
    b_per_w = n // _NW
    n_steps = b_per_w // _CHUNK
    idx3 = (sentence.reshape(-1).astype(jnp.int32)
            .reshape(_NW, n_steps, _HALF, 2)
            .transpose(0, 1, 3, 2)
            .reshape(_NW, n_steps, _CHUNK))
    out = _embed_lookup(idx3, word_embeds)
    return out.reshape(b, s, d)


# speedup vs baseline: 1.1140x; 1.1140x over previous
import functools

import jax
import jax.numpy as jnp
from jax import lax
from jax.experimental import pallas as pl
from jax.experimental.pallas import tpu as pltpu
from jax.experimental.pallas import tpu_sc as plsc

_NC = 2
_NS = 16
_NW = _NC * _NS

_CHUNK = 200
_NBUF = 4


def _embed_lookup(idx3, table128, d):
    nw, n_steps, chunk = idx3.shape
    dp = table128.shape[1]
    b_per_w = n_steps * chunk
    n = nw * b_per_w
    n_groups = n_steps // _NBUF
    mesh = plsc.VectorSubcoreMesh(core_axis_name="c", subcore_axis_name="s")

    @functools.partial(
        pl.kernel,
        mesh=mesh,
        out_type=jax.ShapeDtypeStruct((n, d), jnp.float32),
        scratch_types=[
            pltpu.VMEM((n_steps, chunk), jnp.int32),
            pltpu.VMEM((_NBUF, chunk, dp), jnp.float32),
            pltpu.SemaphoreType.DMA((_NBUF,)),
            pltpu.SemaphoreType.DMA((_NBUF,)),
        ],
        compiler_params=pltpu.CompilerParams(use_tc_tiling_on_sc=False),
    )
    def k(idx_hbm, tab_hbm, out_hbm, idx_all, rows, gsem, ssem):
        wid = lax.axis_index("s") * _NC + lax.axis_index("c")
        base = wid * b_per_w

        def gather_start(b, i):
            pltpu.async_copy(tab_hbm.at[idx_all.at[i]], rows.at[b], gsem.at[b])

        def gather_wait(b, i):
            pltpu.make_async_copy(
                tab_hbm.at[idx_all.at[i]], rows.at[b], gsem.at[b]).wait()

        def _store_desc(b, i):
            return (rows.at[b].at[:, pl.ds(0, d)],
                    out_hbm.at[pl.ds(base + i * chunk, chunk)])

        def store_start(b, i):
            src, dst = _store_desc(b, i)
            pltpu.async_copy(src, dst, ssem.at[b])

        def store_wait(b, i):
            src, dst = _store_desc(b, i)
            pltpu.make_async_copy(src, dst, ssem.at[b]).wait()

        pltpu.sync_copy(idx_hbm.at[wid], idx_all)

        for b in range(_NBUF):
            gather_start(b, b)
        gather_wait(0, 0)
        store_start(0, 0)

        def group(g, carry):
            for b in range(_NBUF):
                i = g * _NBUF + b
                store_wait(b, i - _NBUF)
                gather_start(b, i)
                kk = i - (_NBUF - 1)
                bk = (b + 1) % _NBUF
                gather_wait(bk, kk)
                store_start(bk, kk)
            return carry

        lax.fori_loop(1, n_groups, group, 0, unroll=False)

        for kk in range(n_steps - (_NBUF - 1), n_steps):
            gather_wait(kk % _NBUF, kk)
            store_start(kk % _NBUF, kk)
        for b in range(_NBUF):
            store_wait(b, n_steps - _NBUF + b)

    return k(idx3, table128)


def kernel(sentence, word_embeds):
    b, s = sentence.shape
    d = word_embeds.shape[1]
    n = b * s
    b_per_w = n // _NW
    n_steps = b_per_w // _CHUNK
    idx3 = sentence.reshape(_NW, n_steps, _CHUNK).astype(jnp.int32)
    table128 = jnp.pad(word_embeds, ((0, 0), (0, 128 - d)))
    out = _embed_lookup(idx3, table128, d)
    return out.reshape(b, s, d)


# speedup vs baseline: 1.1140x; 1.0000x over previous
import functools

import jax
import jax.numpy as jnp
from jax import lax
from jax.experimental import pallas as pl
from jax.experimental.pallas import tpu as pltpu
from jax.experimental.pallas import tpu_sc as plsc

_NC = 2
_NS = 16
_NW = _NC * _NS

_CHUNK = 200
_NBUF = 4


def _embed_lookup(idx3, table128, d):
    nw, n_steps, chunk = idx3.shape
    dp = table128.shape[1]
    n_groups = n_steps // _NBUF
    mesh = plsc.VectorSubcoreMesh(core_axis_name="c", subcore_axis_name="s")

    @functools.partial(
        pl.kernel,
        mesh=mesh,
        out_type=jax.ShapeDtypeStruct((nw * n_steps, chunk, d), jnp.float32),
        scratch_types=[
            pltpu.VMEM((n_steps, chunk), jnp.int32),
            pltpu.VMEM((_NBUF, chunk, dp), jnp.float32),
            pltpu.SemaphoreType.DMA((_NBUF,)),
            pltpu.SemaphoreType.DMA((_NBUF,)),
        ],
        compiler_params=pltpu.CompilerParams(use_tc_tiling_on_sc=False),
    )
    def k(idx_hbm, tab_hbm, out_hbm, idx_all, rows, gsem, ssem):
        wid = lax.axis_index("s") * _NC + lax.axis_index("c")
        base = wid * n_steps

        def gather_start(b, i):
            pltpu.async_copy(tab_hbm.at[idx_all.at[i]], rows.at[b], gsem.at[b])

        def gather_wait(b, i):
            pltpu.make_async_copy(
                tab_hbm.at[idx_all.at[i]], rows.at[b], gsem.at[b]).wait()

        def _store_desc(b, i):
            return (rows.at[b].at[:, pl.ds(0, d)], out_hbm.at[base + i])

        def store_start(b, i):
            src, dst = _store_desc(b, i)
            pltpu.async_copy(src, dst, ssem.at[b])

        def store_wait(b, i):
            src, dst = _store_desc(b, i)
            pltpu.make_async_copy(src, dst, ssem.at[b]).wait()

        pltpu.sync_copy(idx_hbm.at[wid], idx_all)

        for b in range(_NBUF):
            gather_start(b, b)
        gather_wait(0, 0)
        store_start(0, 0)

        def group(g, carry):
            for b in range(_NBUF):
                i = g * _NBUF + b
                store_wait(b, i - _NBUF)
                gather_start(b, i)
                kk = i - (_NBUF - 1)
                bk = (b + 1) % _NBUF
                gather_wait(bk, kk)
                store_start(bk, kk)
            return carry

        lax.fori_loop(1, n_groups, group, 0, unroll=False)

        for kk in range(n_steps - (_NBUF - 1), n_steps):
            gather_wait(kk % _NBUF, kk)
            store_start(kk % _NBUF, kk)
        for b in range(_NBUF):
            store_wait(b, n_steps - _NBUF + b)

    return k(idx3, table128)


def kernel(sentence, word_embeds):
    b, s = sentence.shape
    d = word_embeds.shape[1]
    n = b * s
    b_per_w = n // _NW
    n_steps = b_per_w // _CHUNK
    idx3 = sentence.reshape(_NW, n_steps, _CHUNK).astype(jnp.int32)
    table128 = jnp.pad(word_embeds, ((0, 0), (0, 128 - d)))
    out = _embed_lookup(idx3, table128, d)
    return out.reshape(b, s, d)


# speedup vs baseline: 1.1921x; 1.0701x over previous
import functools

import jax
import jax.numpy as jnp
from jax import lax
from jax.experimental import pallas as pl
from jax.experimental.pallas import tpu as pltpu
from jax.experimental.pallas import tpu_sc as plsc

_NC = 2
_NS = 16
_NW = _NC * _NS

_CHUNK = 128
_NBUF = 2


def _embed_lookup(idx2, table2):
    n_rows = idx2.shape[0]
    chunk = idx2.shape[1]
    d = 64
    n_steps = n_rows // _NW
    n = n_rows * chunk
    mesh = plsc.VectorSubcoreMesh(core_axis_name="c", subcore_axis_name="s")

    @functools.partial(
        pl.kernel,
        mesh=mesh,
        out_type=jax.ShapeDtypeStruct((n, d), jnp.float32),
        scratch_types=[
            pltpu.VMEM((n_steps, chunk), jnp.int32),
            pltpu.VMEM((n_steps, chunk), jnp.int32),
            pltpu.VMEM((_NBUF, chunk, 2 * d), jnp.float32),
            pltpu.VMEM((_NBUF, chunk, d), jnp.float32),
            pltpu.SemaphoreType.DMA((_NBUF,)),
            pltpu.SemaphoreType.DMA((_NBUF,)),
        ],
    )
    def k(idx_hbm, tab_hbm, out_hbm, idx_all, ridx, rows2, rows1, gsem, ssem):
        wid = lax.axis_index("s") * _NC + lax.axis_index("c")
        base = wid * n_steps * chunk

        pltpu.sync_copy(idx_hbm.at[pl.ds(wid * n_steps, n_steps)], idx_all)

        def shift_row(r, carry):
            def shift_vec(v, carry2):
                x = idx_all[r, pl.ds(v * 16, 16)]
                ridx[r, pl.ds(v * 16, 16)] = lax.shift_right_logical(x, 1)
                return carry2
            return lax.fori_loop(0, chunk // 16, shift_vec, carry, unroll=4)
        lax.fori_loop(0, n_steps, shift_row, 0, unroll=False)

        def gather_start(b, i):
            pltpu.async_copy(tab_hbm.at[ridx.at[i]], rows2.at[b], gsem.at[b])

        def gather_wait(b, i):
            pltpu.make_async_copy(tab_hbm.at[ridx.at[i]], rows2.at[b],
                                  gsem.at[b]).wait()

        def store_start(b, i):
            pltpu.async_copy(rows1.at[b],
                             out_hbm.at[pl.ds(base + i * chunk, chunk)],
                             ssem.at[b])

        def store_wait(b, i):
            pltpu.make_async_copy(rows1.at[b],
                                  out_hbm.at[pl.ds(base + i * chunk, chunk)],
                                  ssem.at[b]).wait()

        def compact(b, i):
            def blk(v, carry):
                kk0 = v * 16
                hv = lax.bitwise_and(idx_all[i, pl.ds(kk0, 16)], 1)
                offv = hv * d
                for l in range(16):
                    off = offv[l]
                    kr = kk0 + l
                    for m in range(d // 16):
                        rows1[b, kr, pl.ds(m * 16, 16)] = (
                            rows2[b, kr, pl.ds(off + m * 16, 16)])
                return carry
            lax.fori_loop(0, chunk // 16, blk, 0, unroll=False)

        for b in range(_NBUF):
            gather_start(b, b)
        for b in range(_NBUF):
            gather_wait(b, b)
            compact(b, b)
            store_start(b, b)
            gather_start(b, b + _NBUF)

        def step(g, carry):
            for b in range(_NBUF):
                ii = g * _NBUF + b
                gather_wait(b, ii)
                store_wait(b, ii - _NBUF)
                compact(b, ii)
                store_start(b, ii)
                gather_start(b, ii + _NBUF)
            return carry

        lax.fori_loop(1, n_steps // _NBUF - 1, step, 0, unroll=False)

        for b in range(_NBUF):
            ii = n_steps - _NBUF + b
            gather_wait(b, ii)
            store_wait(b, ii - _NBUF)
            compact(b, ii)
            store_start(b, ii)
        for b in range(_NBUF):
            store_wait(b, n_steps - _NBUF + b)

    return k(idx2, table2)


def kernel(sentence, word_embeds):
    b, s = sentence.shape
    d = word_embeds.shape[1]
    n = b * s
    idx2 = sentence.reshape(n // _CHUNK, _CHUNK).astype(jnp.int32)
    table2 = word_embeds.reshape(word_embeds.shape[0] // 2, 2 * d)
    out = _embed_lookup(idx2, table2)
    return out.reshape(b, s, d)


# speedup vs baseline: 1.3721x; 1.1510x over previous
import functools

import jax
import jax.numpy as jnp
from jax import lax
from jax.experimental import pallas as pl
from jax.experimental.pallas import tpu as pltpu
from jax.experimental.pallas import tpu_sc as plsc

_NC = 2
_NS = 16
_NW = _NC * _NS

_CHUNK = 128
_NBUF = 2


def _embed_lookup(idx2, table2):
    n_rows = idx2.shape[0]
    chunk = idx2.shape[1]
    d = 64
    n_steps = n_rows // _NW
    n = n_rows * chunk
    mesh = plsc.VectorSubcoreMesh(core_axis_name="c", subcore_axis_name="s")

    @functools.partial(
        pl.kernel,
        mesh=mesh,
        out_type=jax.ShapeDtypeStruct((n, d), jnp.float32),
        scratch_types=[
            pltpu.VMEM((n_steps, chunk), jnp.int32),
            pltpu.VMEM((_NBUF, chunk, 2 * d), jnp.float32),
            pltpu.VMEM((_NBUF, chunk, d), jnp.float32),
            pltpu.SemaphoreType.DMA((_NBUF,)),
            pltpu.SemaphoreType.DMA((_NBUF,)),
        ],
    )
    def k(idx_hbm, tab_hbm, out_hbm, idx_all, rows2, rows1, gsem, ssem):
        wid = lax.axis_index("s") * _NC + lax.axis_index("c")
        base = wid * n_steps * chunk

        pltpu.sync_copy(idx_hbm.at[pl.ds(wid * n_steps, n_steps)], idx_all)

        def gather_start(b, i):
            pltpu.async_copy(tab_hbm.at[idx_all.at[i]], rows2.at[b],
                             gsem.at[b])

        def gather_wait(b, i):
            pltpu.make_async_copy(tab_hbm.at[idx_all.at[i]], rows2.at[b],
                                  gsem.at[b]).wait()

        def store_start(b, i):
            pltpu.async_copy(rows1.at[b],
                             out_hbm.at[pl.ds(base + i * chunk, chunk)],
                             ssem.at[b])

        def store_wait(b, i):
            pltpu.make_async_copy(rows1.at[b],
                                  out_hbm.at[pl.ds(base + i * chunk, chunk)],
                                  ssem.at[b]).wait()

        def compact(b, i):
            def blk(v, carry):
                kk0 = v * 16
                for l in range(16):
                    kr = kk0 + l
                    for m in range(d // 16):
                        rows1[b, kr, pl.ds(m * 16, 16)] = (
                            rows2[b, kr, pl.ds(m * 16, 16)])
                return carry
            lax.fori_loop(0, chunk // 16, blk, 0, unroll=2)

        for b in range(_NBUF):
            gather_start(b, b)
        for b in range(_NBUF):
            gather_wait(b, b)
            compact(b, b)
            store_start(b, b)
            gather_start(b, b + _NBUF)

        def step(g, carry):
            for b in range(_NBUF):
                ii = g * _NBUF + b
                gather_wait(b, ii)
                store_wait(b, ii - _NBUF)
                compact(b, ii)
                store_start(b, ii)
                gather_start(b, ii + _NBUF)
            return carry

        lax.fori_loop(1, n_steps // _NBUF - 1, step, 0, unroll=False)

        for b in range(_NBUF):
            ii = n_steps - _NBUF + b
            gather_wait(b, ii)
            store_wait(b, ii - _NBUF)
            compact(b, ii)
            store_start(b, ii)
        for b in range(_NBUF):
            store_wait(b, n_steps - _NBUF + b)

    return k(idx2, table2)


def kernel(sentence, word_embeds):
    b, s = sentence.shape
    d = word_embeds.shape[1]
    n = b * s
    idx2 = sentence.reshape(n // _CHUNK, _CHUNK).astype(jnp.int32)
    table2 = jnp.pad(word_embeds, ((0, 0), (0, d)))
    out = _embed_lookup(idx2, table2)
    return out.reshape(b, s, d)


# speedup vs baseline: 1.3735x; 1.0010x over previous
import functools

import jax
import jax.numpy as jnp
from jax import lax
from jax.experimental import pallas as pl
from jax.experimental.pallas import tpu as pltpu
from jax.experimental.pallas import tpu_sc as plsc

_NC = 2
_NS = 16
_NW = _NC * _NS

_CHUNK = 128
_NBUF = 2


def _embed_lookup(idx2, table2):
    n_rows = idx2.shape[0]
    chunk = idx2.shape[1]
    d = 64
    n_steps = n_rows // _NW
    n = n_rows * chunk
    mesh = plsc.VectorSubcoreMesh(core_axis_name="c", subcore_axis_name="s")

    @functools.partial(
        pl.kernel,
        mesh=mesh,
        out_type=jax.ShapeDtypeStruct((n, d), jnp.float32),
        scratch_types=[
            pltpu.VMEM((n_steps, chunk), jnp.int32),
            pltpu.VMEM((_NBUF, chunk, 2 * d), jnp.float32),
            pltpu.VMEM((_NBUF, chunk, d), jnp.float32),
            pltpu.SemaphoreType.DMA((_NBUF,)),
            pltpu.SemaphoreType.DMA((_NBUF,)),
        ],
    )
    def k(idx_hbm, tab_hbm, out_hbm, idx_all, rows2, rows1, gsem, ssem):
        wid = lax.axis_index("s") * _NC + lax.axis_index("c")
        base = wid * n_steps * chunk

        pltpu.sync_copy(idx_hbm.at[pl.ds(wid * n_steps, n_steps)], idx_all)

        def gather_start(b, i):
            pltpu.async_copy(tab_hbm.at[idx_all.at[i]], rows2.at[b],
                             gsem.at[b])

        def gather_wait(b, i):
            pltpu.make_async_copy(tab_hbm.at[idx_all.at[i]], rows2.at[b],
                                  gsem.at[b]).wait()

        def store_start(b, i):
            pltpu.async_copy(rows1.at[b],
                             out_hbm.at[pl.ds(base + i * chunk, chunk)],
                             ssem.at[b])

        def store_wait(b, i):
            pltpu.make_async_copy(rows1.at[b],
                                  out_hbm.at[pl.ds(base + i * chunk, chunk)],
                                  ssem.at[b]).wait()

        def compact(b, i):
            def blk(v, carry):
                kk0 = v * 16
                for l in range(16):
                    kr = kk0 + l
                    for m in range(d // 16):
                        rows1[b, kr, pl.ds(m * 16, 16)] = (
                            rows2[b, kr, pl.ds(m * 16, 16)])
                return carry
            lax.fori_loop(0, chunk // 16, blk, 0, unroll=4)

        for b in range(_NBUF):
            gather_start(b, b)
        for b in range(_NBUF):
            gather_wait(b, b)
            compact(b, b)
            store_start(b, b)
            gather_start(b, b + _NBUF)

        def step(g, carry):
            for b in range(_NBUF):
                ii = g * _NBUF + b
                gather_wait(b, ii)
                store_wait(b, ii - _NBUF)
                compact(b, ii)
                store_start(b, ii)
                gather_start(b, ii + _NBUF)
            return carry

        lax.fori_loop(1, n_steps // _NBUF - 1, step, 0, unroll=False)

        for b in range(_NBUF):
            ii = n_steps - _NBUF + b
            gather_wait(b, ii)
            store_wait(b, ii - _NBUF)
            compact(b, ii)
            store_start(b, ii)
        for b in range(_NBUF):
            store_wait(b, n_steps - _NBUF + b)

    return k(idx2, table2)


def kernel(sentence, word_embeds):
    b, s = sentence.shape
    d = word_embeds.shape[1]
    n = b * s
    idx2 = sentence.reshape(n // _CHUNK, _CHUNK).astype(jnp.int32)
    table2 = jnp.pad(word_embeds, ((0, 0), (0, d)))
    out = _embed_lookup(idx2, table2)
    return out.reshape(b, s, d)
